# baseline (speedup 1.0000x reference)
import functools

import jax
import jax.numpy as jnp
from jax import lax
from jax.experimental import pallas as pl
from jax.experimental.pallas import tpu as pltpu
from jax.experimental.pallas import tpu_sc as plsc

NC = 2
NS = 16
B_GRAPHS = 64

F32 = jnp.float32
I32 = jnp.int32


def _divisor_le(n, cap):
    for d in range(cap, 0, -1):
        if n % d == 0:
            return d
    return 1


def _make_deg_kernel(n_rows, N_pad, KG=33):
    rows_per_w = n_rows // (NC * NS)
    G = rows_per_w // KG
    assert rows_per_w % KG == 0 and G % 2 == 0
    npt = N_pad // NS
    mesh = plsc.VectorSubcoreMesh(core_axis_name="c", subcore_axis_name="s",
                                  num_cores=NC, num_subcores=NS)

    @functools.partial(
        pl.kernel,
        out_type=jax.ShapeDtypeStruct((NC * N_pad,), F32),
        mesh=mesh,
        scratch_types=[
            pltpu.VMEM((npt,), F32),
            pltpu.VMEM_SHARED((N_pad,), F32),
            pltpu.SemaphoreType.DMA,
            pltpu.SemaphoreType.DMA,
        ] + [pltpu.VMEM((KG, 3, 128), I32) for _ in range(2)]
          + [pltpu.VMEM((128,), F32) for _ in range(2 * KG)]
          + [pltpu.VMEM((128,), I32) for _ in range(2 * KG)],
        compiler_params=pltpu.CompilerParams(use_tc_tiling_on_sc=False,
                                             needs_layout_passes=False,
                                             skip_device_barrier=True),
    )
    def deg_kernel(cmb_hbm, out_hbm, z_v, deg_sh, sem_l, sem_s, *bufs):
        cmb_v = bufs[0:2]
        wbufs = [bufs[2 + b * KG: 2 + (b + 1) * KG] for b in range(2)]
        ibufs = [bufs[2 + 2 * KG + b * KG: 2 + 2 * KG + (b + 1) * KG]
                 for b in range(2)]
        c = lax.axis_index("c")
        s = lax.axis_index("s")

        def zb(i, carry):
            z_v[pl.ds(i * 16, 16)] = jnp.zeros((16,), F32)
            return carry
        lax.fori_loop(0, npt // 16, zb, 0)
        pltpu.sync_copy(z_v, deg_sh.at[pl.ds(s * npt, npt)])
        plsc.subcore_barrier()

        base = (s * NC + c) * rows_per_w

        def load_cmb(g, b, sync):
            r0 = base + g * KG
            if sync:
                pltpu.sync_copy(cmb_hbm.at[pl.ds(r0, KG)], cmb_v[b])
            else:
                pltpu.async_copy(cmb_hbm.at[pl.ds(r0, KG)], cmb_v[b], sem_l)

        def wait_cmb(b):
            pltpu.make_async_copy(cmb_hbm.at[pl.ds(0, KG)], cmb_v[b],
                                  sem_l).wait()

        def stage(b):
            for j in range(KG):
                def wb(i, carry):
                    wbufs[b][j][pl.ds(i * 16, 16)] = plsc.bitcast(
                        cmb_v[b][j, 2, pl.ds(i * 16, 16)], F32)
                    ibufs[b][j][pl.ds(i * 16, 16)] = (
                        cmb_v[b][j, 0, pl.ds(i * 16, 16)])
                    return carry
                lax.fori_loop(0, 8, wb, 0)

        def fire_scatters(b):
            for j in range(KG):
                pltpu.async_copy(wbufs[b][j], deg_sh.at[ibufs[b][j]],
                                 sem_s, add=True)

        def wait_scatters(b):
            for j in range(KG):
                pltpu.make_async_copy(wbufs[b][j], deg_sh.at[ibufs[b][j]],
                                      sem_s).wait()

        load_cmb(0, 0, True)

        def gb(t, carry):
            for b in range(2):
                g = 2 * t + b

                @pl.when(g + 1 < G)
                def _():
                    load_cmb(g + 1, 1 - b, False)
                stage(b)
                @pl.when(g >= 1)
                def _():
                    wait_scatters(1 - b)
                fire_scatters(b)

                @pl.when(g + 1 < G)
                def _():
                    wait_cmb(1 - b)
            return carry
        lax.fori_loop(0, G // 2, gb, 0)
        wait_scatters(1)

        plsc.subcore_barrier()
        pltpu.sync_copy(deg_sh.at[pl.ds(s * npt, npt)], z_v)
        pltpu.sync_copy(z_v, out_hbm.at[pl.ds(c * N_pad + s * npt, npt)])

    return deg_kernel


BF16 = jnp.bfloat16


def _make_agg_kernel(n_rows, N_pad, D, KG=2):
    rows_per_w = n_rows // (NC * NS)
    G = rows_per_w // KG
    assert G % 3 == 0 and G > 3
    npt = N_pad // NS
    mesh = plsc.VectorSubcoreMesh(core_axis_name="c", subcore_axis_name="s",
                                  num_cores=NC, num_subcores=NS)

    NR = 3

    @functools.partial(
        pl.kernel,
        out_type=jax.ShapeDtypeStruct((NC, N_pad, D), BF16),
        mesh=mesh,
        scratch_types=[
            pltpu.VMEM_SHARED((N_pad, D), BF16),
            pltpu.SemaphoreType.DMA,
            pltpu.SemaphoreType.DMA,
            pltpu.SemaphoreType.DMA,
        ] + [pltpu.VMEM((KG, 3, 128), I32) for _ in range(NR)]
          + [pltpu.VMEM((KG, 128, D), BF16) for _ in range(NR)]
          + [pltpu.VMEM((128,), I32) for _ in range(NR * KG)],
        compiler_params=pltpu.CompilerParams(use_tc_tiling_on_sc=False,
                                             needs_layout_passes=False,
                                             skip_device_barrier=True),
    )
    def agg_kernel(cmb_hbm, tbl_hbm, out_hbm, acc_sh, sem_g, sem_s, sem_l,
                   *ring):
        cmb_v = ring[0:NR]
        rows_v = ring[NR:2 * NR]
        sbufs = [ring[2 * NR + b * KG: 2 * NR + (b + 1) * KG]
                 for b in range(NR)]
        c = lax.axis_index("c")
        s = lax.axis_index("s")

        z32 = jnp.zeros((32,), BF16)

        def zb(i, carry):
            for f0 in range(0, D, 32):
                rows_v[0][0, i, f0:f0 + 32] = z32
            return carry
        lax.fori_loop(0, 128, zb, 0)
        nfull, rem = npt // 128, npt % 128

        def zc(i, carry):
            pltpu.sync_copy(rows_v[0].at[0],
                            acc_sh.at[pl.ds(s * npt + i * 128, 128)])
            return carry
        lax.fori_loop(0, nfull, zc, 0)
        if rem:
            pltpu.sync_copy(rows_v[0].at[0, pl.ds(0, rem)],
                            acc_sh.at[pl.ds(s * npt + nfull * 128, rem)])
        plsc.subcore_barrier()

        base = (s * NC + c) * rows_per_w

        def load_cmb(g, q, sync):
            r0 = base + g * KG
            if sync:
                pltpu.sync_copy(cmb_hbm.at[pl.ds(r0, KG)], cmb_v[q])
            else:
                pltpu.async_copy(cmb_hbm.at[pl.ds(r0, KG)], cmb_v[q], sem_l)

        def wait_cmb(q):
            pltpu.make_async_copy(cmb_hbm.at[pl.ds(0, KG)], cmb_v[q],
                                  sem_l).wait()

        def stage_and_fire(q):
            for j in range(KG):
                def cb(i, carry):
                    sbufs[q][j][pl.ds(i * 16, 16)] = (
                        cmb_v[q][j, 1, pl.ds(i * 16, 16)])
                    return carry
                lax.fori_loop(0, 8, cb, 0)
            for j in range(KG):
                pltpu.async_copy(tbl_hbm.at[cmb_v[q].at[j, 0]],
                                 rows_v[q].at[j], sem_g)

        def wait_gathers(q):
            for j in range(KG):
                pltpu.make_async_copy(tbl_hbm.at[cmb_v[q].at[j, 0]],
                                      rows_v[q].at[j], sem_g).wait()

        def scale(q):
            for j in range(KG):
                def sb(i, carry):
                    w16 = plsc.bitcast(cmb_v[q][j, 2, pl.ds(i * 16, 16)], F32)
                    for l in range(16):
                        e = i * 16 + l
                        wf = lax.broadcast_in_dim(w16[l], (16,), ())
                        w = plsc.pack(wf, wf,
                                      format=plsc.PackFormat.INTERLEAVED)
                        for f0 in range(0, D, 32):
                            rows_v[q][j, e, f0:f0 + 32] = (
                                rows_v[q][j, e, f0:f0 + 32] * w)
                    return carry
                lax.fori_loop(0, 8, sb, 0)

        def fire_scatters(q):
            for j in range(KG):
                pltpu.async_copy(rows_v[q].at[j], acc_sh.at[sbufs[q][j]],
                                 sem_s, add=True)

        def wait_scatters(q):
            for j in range(KG):
                pltpu.make_async_copy(rows_v[q].at[j], acc_sh.at[sbufs[q][j]],
                                      sem_s).wait()

        load_cmb(0, 0, True)
        stage_and_fire(0)
        load_cmb(1, 1, True)
        stage_and_fire(1)
        if G > 2:
            load_cmb(2, 2, False)

        def gb(t, carry):
            for b in range(NR):
                g = NR * t + b
                wait_gathers(b)
                scale(b)
                fire_scatters(b)

                @pl.when(g + 3 < G)
                def _():
                    load_cmb(g + 3, b, False)

                @pl.when(g >= 1)
                def _():
                    wait_scatters((b + 1) % NR)

                @pl.when(g + 2 < G)
                def _():
                    wait_cmb((b + 2) % NR)
                    stage_and_fire((b + 2) % NR)
            return carry
        lax.fori_loop(0, G // NR, gb, 0)
        wait_scatters((G - 1) % NR)

        plsc.subcore_barrier()
        def oc(i, carry):
            pltpu.sync_copy(acc_sh.at[pl.ds(s * npt + i * 128, 128)],
                            rows_v[0].at[0])
            pltpu.sync_copy(rows_v[0].at[0],
                            out_hbm.at[c, pl.ds(s * npt + i * 128, 128)])
            return carry
        lax.fori_loop(0, nfull, oc, 0)
        if rem:
            pltpu.sync_copy(acc_sh.at[pl.ds(s * npt + nfull * 128, rem)],
                            rows_v[0].at[0, pl.ds(0, rem)])
            pltpu.sync_copy(rows_v[0].at[0, pl.ds(0, rem)],
                            out_hbm.at[c, pl.ds(s * npt + nfull * 128, rem)])

    return agg_kernel


def _make_tc1(N_pad, R, D_IN, D_HID, DP):
    NB = N_pad // R
    D = D_HID // 2

    def body(x_ref, degp_ref, w0_ref, w1_ref, xw0_ref, tbl_ref):
        x = x_ref[...]
        deg = degp_ref[0] + degp_ref[1]
        dis = jnp.where(deg > 0.0,
                        lax.rsqrt(jnp.maximum(deg, 1e-30)), 0.0)
        xw0_ref[...] = jnp.dot(x, w0_ref[...], preferred_element_type=F32)
        t = jnp.dot(x * dis, w1_ref[...], preferred_element_type=F32)
        zpad = jnp.zeros((R, DP - D_HID), F32)
        tbl_ref[...] = jnp.concatenate([t, zpad], axis=1).astype(BF16)

    return pl.pallas_call(
        body,
        grid=(NB,),
        in_specs=[
            pl.BlockSpec((R, D_IN), lambda i: (i, 0)),
            pl.BlockSpec((2, R, 1), lambda i: (0, i, 0)),
            pl.BlockSpec((D_IN, D_HID), lambda i: (0, 0)),
            pl.BlockSpec((D_IN, D_HID), lambda i: (0, 0)),
        ],
        out_specs=[
            pl.BlockSpec((R, D_HID), lambda i: (i, 0)),
            pl.BlockSpec((R, DP), lambda i: (i, 0)),
        ],
        out_shape=[
            jax.ShapeDtypeStruct((N_pad, D_HID), F32),
            jax.ShapeDtypeStruct((N_pad, DP), BF16),
        ],
    )


def _make_tc2(N_pad, R, D_HID, D_RNN, N_CLS, DP):
    NB = N_pad // R
    D = D_HID // 2

    def body(xw0_ref, agg_ref, degp_ref, bch_ref, batch_ref,
             wih_ref, whh_ref, bih_ref, bhh_ref, wfc_ref, bfc_ref,
             out_ref, pooled_acc, pre_scr, outs_scr):
        i = pl.program_id(0)

        @pl.when(i == 0)
        def _init():
            pooled_acc[...] = jnp.zeros_like(pooled_acc)

        deg = degp_ref[0] + degp_ref[1]
        dis = jnp.where(deg > 0.0,
                        lax.rsqrt(jnp.maximum(deg, 1e-30)), 0.0)
        ag = agg_ref[0].astype(F32) + agg_ref[1].astype(F32)
        agg = ag[:, :D_HID]
        h = xw0_ref[...] + bch_ref[...] - dis * agg
        h = jnp.maximum(h, 0.0)

        bids = batch_ref[...]
        onehot = (bids == lax.broadcasted_iota(I32, (R, B_GRAPHS), 1)
                  ).astype(F32)
        pooled_acc[...] += lax.dot_general(
            onehot, h, (((0,), (0,)), ((), ())), preferred_element_type=F32)

        @pl.when(i == NB - 1)
        def _tail():
            pooled = pooled_acc[...]
            pre = lax.dot_general(
                pooled, wih_ref[...], (((1,), (1,)), ((), ())),
                preferred_element_type=F32) + bih_ref[...] + bhh_ref[...]
            pre_scr[...] = pre

            def step(t, hprev):
                z = pre_scr[pl.ds(t, 1), :] + lax.dot_general(
                    hprev, whh_ref[...], (((1,), (1,)), ((), ())),
                    preferred_element_type=F32)
                hn = jnp.tanh(z)
                outs_scr[pl.ds(t, 1), :] = hn
                return hn
            lax.fori_loop(0, B_GRAPHS, step, jnp.zeros((1, D_RNN), F32))

            logits = lax.dot_general(
                outs_scr[...], wfc_ref[...], (((1,), (1,)), ((), ())),
                preferred_element_type=F32) + bfc_ref[...]
            m = jnp.max(logits, axis=1, keepdims=True)
            sh = logits - m
            out_ref[...] = sh - jnp.log(
                jnp.sum(jnp.exp(sh), axis=1, keepdims=True))

    return pl.pallas_call(
        body,
        grid=(NB,),
        in_specs=[
            pl.BlockSpec((R, D_HID), lambda i: (i, 0)),
            pl.BlockSpec((2, R, DP), lambda i: (0, i, 0)),
            pl.BlockSpec((2, R, 1), lambda i: (0, i, 0)),
            pl.BlockSpec((1, D_HID), lambda i: (0, 0)),
            pl.BlockSpec((R, 1), lambda i: (i, 0)),
            pl.BlockSpec((D_RNN, D_HID), lambda i: (0, 0)),
            pl.BlockSpec((D_RNN, D_RNN), lambda i: (0, 0)),
            pl.BlockSpec((1, D_RNN), lambda i: (0, 0)),
            pl.BlockSpec((1, D_RNN), lambda i: (0, 0)),
            pl.BlockSpec((N_CLS, D_RNN), lambda i: (0, 0)),
            pl.BlockSpec((1, N_CLS), lambda i: (0, 0)),
        ],
        out_specs=pl.BlockSpec((B_GRAPHS, N_CLS), lambda i: (0, 0)),
        out_shape=jax.ShapeDtypeStruct((B_GRAPHS, N_CLS), F32),
        scratch_shapes=[
            pltpu.VMEM((B_GRAPHS, D_HID), F32),
            pltpu.VMEM((B_GRAPHS, D_RNN), F32),
            pltpu.VMEM((B_GRAPHS, D_RNN), F32),
        ],
    )


def kernel(x, edge_index, edge_weight, batch, batch_size,
           W0, W1, b_cheb, W_ih, W_hh, b_ih, b_hh, W_fc, b_fc):
    N, D_IN = x.shape
    E = edge_index.shape[1]
    D_HID = W0.shape[1]
    D_RNN = W_ih.shape[0]
    N_CLS = W_fc.shape[0]
    DP = 64

    R = 1024
    N_pad = -(-N // R) * R
    n_rows = -(-E // (128 * NC * NS * 6)) * (NC * NS * 6)
    E_pad = n_rows * 128

    row = edge_index[0].astype(I32)
    col = edge_index[1].astype(I32)
    pad_e = E_pad - E
    row_p = jnp.pad(row, (0, pad_e)).reshape(n_rows, 128)
    col_p = jnp.pad(col, (0, pad_e)).reshape(n_rows, 128)
    ew_p = jnp.pad(edge_weight, (0, pad_e)).reshape(n_rows, 128)
    ew_bits = lax.bitcast_convert_type(ew_p, I32)
    cmb = jnp.stack([row_p, col_p, ew_bits], axis=1)

    degp = _make_deg_kernel(n_rows, N_pad)(cmb)
    degp3 = degp.reshape(NC, N_pad, 1)

    x_p = jnp.pad(x, ((0, N_pad - N), (0, 0)))
    xw0, tbl = _make_tc1(N_pad, R, D_IN, D_HID, DP)(x_p, degp3, W0, W1)

    aggs = _make_agg_kernel(n_rows, N_pad, DP)(cmb, tbl)

    batch_p = jnp.pad(batch.astype(I32), (0, N_pad - N),
                      constant_values=B_GRAPHS).reshape(N_pad, 1)
    out = _make_tc2(N_pad, R, D_HID, D_RNN, N_CLS, DP)(
        xw0, aggs, degp3, b_cheb.reshape(1, D_HID), batch_p,
        W_ih, W_hh, b_ih.reshape(1, D_RNN), b_hh.reshape(1, D_RNN),
        W_fc, b_fc.reshape(1, N_CLS))
    return out


# speedup vs baseline: 1.0414x; 1.0414x over previous
import functools

import jax
import jax.numpy as jnp
from jax import lax
from jax.experimental import pallas as pl
from jax.experimental.pallas import tpu as pltpu
from jax.experimental.pallas import tpu_sc as plsc

NC = 2
NS = 16
B_GRAPHS = 64

F32 = jnp.float32
I32 = jnp.int32


def _divisor_le(n, cap):
    for d in range(cap, 0, -1):
        if n % d == 0:
            return d
    return 1


def _make_deg_kernel(n_rows, N_pad, KG=33):
    rows_per_w = n_rows // (NC * NS)
    G = rows_per_w // KG
    assert rows_per_w % KG == 0 and G % 2 == 0
    npt = N_pad // NS
    mesh = plsc.VectorSubcoreMesh(core_axis_name="c", subcore_axis_name="s",
                                  num_cores=NC, num_subcores=NS)

    @functools.partial(
        pl.kernel,
        out_type=jax.ShapeDtypeStruct((NC * N_pad,), F32),
        mesh=mesh,
        scratch_types=[
            pltpu.VMEM((npt,), F32),
            pltpu.VMEM_SHARED((N_pad,), F32),
            pltpu.SemaphoreType.DMA,
            pltpu.SemaphoreType.DMA,
        ] + [pltpu.VMEM((KG, 3, 128), I32) for _ in range(2)]
          + [pltpu.VMEM((128,), F32) for _ in range(2 * KG)]
          + [pltpu.VMEM((128,), I32) for _ in range(2 * KG)],
        compiler_params=pltpu.CompilerParams(use_tc_tiling_on_sc=False,
                                             needs_layout_passes=False,
                                             skip_device_barrier=True),
    )
    def deg_kernel(cmb_hbm, out_hbm, z_v, deg_sh, sem_l, sem_s, *bufs):
        cmb_v = bufs[0:2]
        wbufs = [bufs[2 + b * KG: 2 + (b + 1) * KG] for b in range(2)]
        ibufs = [bufs[2 + 2 * KG + b * KG: 2 + 2 * KG + (b + 1) * KG]
                 for b in range(2)]
        c = lax.axis_index("c")
        s = lax.axis_index("s")

        def zb(i, carry):
            z_v[pl.ds(i * 16, 16)] = jnp.zeros((16,), F32)
            return carry
        lax.fori_loop(0, npt // 16, zb, 0)
        pltpu.sync_copy(z_v, deg_sh.at[pl.ds(s * npt, npt)])
        plsc.subcore_barrier()

        base = (s * NC + c) * rows_per_w

        def load_cmb(g, b, sync):
            r0 = base + g * KG
            if sync:
                pltpu.sync_copy(cmb_hbm.at[pl.ds(r0, KG)], cmb_v[b])
            else:
                pltpu.async_copy(cmb_hbm.at[pl.ds(r0, KG)], cmb_v[b], sem_l)

        def wait_cmb(b):
            pltpu.make_async_copy(cmb_hbm.at[pl.ds(0, KG)], cmb_v[b],
                                  sem_l).wait()

        def stage(b):
            for j in range(KG):
                def wb(i, carry):
                    wbufs[b][j][pl.ds(i * 16, 16)] = plsc.bitcast(
                        cmb_v[b][j, 2, pl.ds(i * 16, 16)], F32)
                    ibufs[b][j][pl.ds(i * 16, 16)] = (
                        cmb_v[b][j, 0, pl.ds(i * 16, 16)])
                    return carry
                lax.fori_loop(0, 8, wb, 0)

        def fire_scatters(b):
            for j in range(KG):
                pltpu.async_copy(wbufs[b][j], deg_sh.at[ibufs[b][j]],
                                 sem_s, add=True)

        def wait_scatters(b):
            for j in range(KG):
                pltpu.make_async_copy(wbufs[b][j], deg_sh.at[ibufs[b][j]],
                                      sem_s).wait()

        load_cmb(0, 0, True)

        def gb(t, carry):
            for b in range(2):
                g = 2 * t + b

                @pl.when(g + 1 < G)
                def _():
                    load_cmb(g + 1, 1 - b, False)
                stage(b)
                @pl.when(g >= 1)
                def _():
                    wait_scatters(1 - b)
                fire_scatters(b)

                @pl.when(g + 1 < G)
                def _():
                    wait_cmb(1 - b)
            return carry
        lax.fori_loop(0, G // 2, gb, 0)
        wait_scatters(1)

        plsc.subcore_barrier()
        pltpu.sync_copy(deg_sh.at[pl.ds(s * npt, npt)], z_v)
        pltpu.sync_copy(z_v, out_hbm.at[pl.ds(c * N_pad + s * npt, npt)])

    return deg_kernel


BF16 = jnp.bfloat16


def _make_agg_kernel(n_rows, N_pad, D, KG=2, split=(2, 1)):
    tot = split[0] + split[1]
    rows_c0 = (n_rows // NS) * split[0] // tot
    rows_c0 -= rows_c0 % (3 * KG)
    rows_c1 = n_rows // NS - rows_c0
    assert rows_c1 % (3 * KG) == 0 and rows_c0 > 0 and rows_c1 > 0
    G0, G1 = rows_c0 // KG, rows_c1 // KG
    npt = N_pad // NS
    mesh = plsc.VectorSubcoreMesh(core_axis_name="c", subcore_axis_name="s",
                                  num_cores=NC, num_subcores=NS)

    NR = 3

    @functools.partial(
        pl.kernel,
        out_type=jax.ShapeDtypeStruct((NC, N_pad, D), BF16),
        mesh=mesh,
        scratch_types=[
            pltpu.VMEM_SHARED((N_pad, D), BF16),
            pltpu.SemaphoreType.DMA,
            pltpu.SemaphoreType.DMA,
            pltpu.SemaphoreType.DMA,
        ] + [pltpu.VMEM((KG, 3, 128), I32) for _ in range(NR)]
          + [pltpu.VMEM((KG, 128, D), BF16) for _ in range(NR)]
          + [pltpu.VMEM((128,), I32) for _ in range(NR * KG)],
        compiler_params=pltpu.CompilerParams(use_tc_tiling_on_sc=False,
                                             needs_layout_passes=False,
                                             skip_device_barrier=True),
    )
    def agg_kernel(cmb_hbm, tbl_hbm, out_hbm, acc_sh, sem_g, sem_s, sem_l,
                   *ring):
        cmb_v = ring[0:NR]
        rows_v = ring[NR:2 * NR]
        sbufs = [ring[2 * NR + b * KG: 2 * NR + (b + 1) * KG]
                 for b in range(NR)]
        c = lax.axis_index("c")
        s = lax.axis_index("s")

        z32 = jnp.zeros((32,), BF16)

        def zb(i, carry):
            for f0 in range(0, D, 32):
                rows_v[0][0, i, f0:f0 + 32] = z32
            return carry
        lax.fori_loop(0, 128, zb, 0)
        nfull, rem = npt // 128, npt % 128

        def zc(i, carry):
            pltpu.sync_copy(rows_v[0].at[0],
                            acc_sh.at[pl.ds(s * npt + i * 128, 128)])
            return carry
        lax.fori_loop(0, nfull, zc, 0)
        if rem:
            pltpu.sync_copy(rows_v[0].at[0, pl.ds(0, rem)],
                            acc_sh.at[pl.ds(s * npt + nfull * 128, rem)])
        plsc.subcore_barrier()

        base = jnp.where(c == 0, s * rows_c0, NS * rows_c0 + s * rows_c1)
        G = jnp.where(c == 0, G0, G1)

        def load_cmb(g, q, sync):
            r0 = base + g * KG
            if sync:
                pltpu.sync_copy(cmb_hbm.at[pl.ds(r0, KG)], cmb_v[q])
            else:
                pltpu.async_copy(cmb_hbm.at[pl.ds(r0, KG)], cmb_v[q], sem_l)

        def wait_cmb(q):
            pltpu.make_async_copy(cmb_hbm.at[pl.ds(0, KG)], cmb_v[q],
                                  sem_l).wait()

        def stage_and_fire(q):
            for j in range(KG):
                def cb(i, carry):
                    sbufs[q][j][pl.ds(i * 16, 16)] = (
                        cmb_v[q][j, 1, pl.ds(i * 16, 16)])
                    return carry
                lax.fori_loop(0, 8, cb, 0)
            for j in range(KG):
                pltpu.async_copy(tbl_hbm.at[cmb_v[q].at[j, 0]],
                                 rows_v[q].at[j], sem_g)

        def wait_gathers(q):
            for j in range(KG):
                pltpu.make_async_copy(tbl_hbm.at[cmb_v[q].at[j, 0]],
                                      rows_v[q].at[j], sem_g).wait()

        def scale(q):
            for j in range(KG):
                def sb(i, carry):
                    w16 = plsc.bitcast(cmb_v[q][j, 2, pl.ds(i * 16, 16)], F32)
                    for l in range(16):
                        e = i * 16 + l
                        wf = lax.broadcast_in_dim(w16[l], (16,), ())
                        w = plsc.pack(wf, wf,
                                      format=plsc.PackFormat.INTERLEAVED)
                        for f0 in range(0, D, 32):
                            rows_v[q][j, e, f0:f0 + 32] = (
                                rows_v[q][j, e, f0:f0 + 32] * w)
                    return carry
                lax.fori_loop(0, 8, sb, 0)

        def fire_scatters(q):
            for j in range(KG):
                pltpu.async_copy(rows_v[q].at[j], acc_sh.at[sbufs[q][j]],
                                 sem_s, add=True)

        def wait_scatters(q):
            for j in range(KG):
                pltpu.make_async_copy(rows_v[q].at[j], acc_sh.at[sbufs[q][j]],
                                      sem_s).wait()

        load_cmb(0, 0, True)
        stage_and_fire(0)
        load_cmb(1, 1, True)
        stage_and_fire(1)
        load_cmb(2, 2, False)

        def gb(t, carry):
            for b in range(NR):
                g = NR * t + b
                wait_gathers(b)
                scale(b)
                fire_scatters(b)

                @pl.when(g + 3 < G)
                def _():
                    load_cmb(g + 3, b, False)

                @pl.when(g >= 1)
                def _():
                    wait_scatters((b + 1) % NR)

                @pl.when(g + 2 < G)
                def _():
                    wait_cmb((b + 2) % NR)
                    stage_and_fire((b + 2) % NR)
            return carry
        lax.fori_loop(0, G // NR, gb, 0)
        wait_scatters(NR - 1)

        plsc.subcore_barrier()
        def oc(i, carry):
            pltpu.sync_copy(acc_sh.at[pl.ds(s * npt + i * 128, 128)],
                            rows_v[0].at[0])
            pltpu.sync_copy(rows_v[0].at[0],
                            out_hbm.at[c, pl.ds(s * npt + i * 128, 128)])
            return carry
        lax.fori_loop(0, nfull, oc, 0)
        if rem:
            pltpu.sync_copy(acc_sh.at[pl.ds(s * npt + nfull * 128, rem)],
                            rows_v[0].at[0, pl.ds(0, rem)])
            pltpu.sync_copy(rows_v[0].at[0, pl.ds(0, rem)],
                            out_hbm.at[c, pl.ds(s * npt + nfull * 128, rem)])

    return agg_kernel


def _make_tc1(N_pad, R, D_IN, D_HID, DP):
    NB = N_pad // R
    D = D_HID // 2

    def body(x_ref, degp_ref, w0_ref, w1_ref, xw0_ref, tbl_ref):
        x = x_ref[...]
        deg = degp_ref[0] + degp_ref[1]
        dis = jnp.where(deg > 0.0,
                        lax.rsqrt(jnp.maximum(deg, 1e-30)), 0.0)
        xw0_ref[...] = jnp.dot(x, w0_ref[...], preferred_element_type=F32)
        t = jnp.dot(x * dis, w1_ref[...], preferred_element_type=F32)
        zpad = jnp.zeros((R, DP - D_HID), F32)
        tbl_ref[...] = jnp.concatenate([t, zpad], axis=1).astype(BF16)

    return pl.pallas_call(
        body,
        grid=(NB,),
        in_specs=[
            pl.BlockSpec((R, D_IN), lambda i: (i, 0)),
            pl.BlockSpec((2, R, 1), lambda i: (0, i, 0)),
            pl.BlockSpec((D_IN, D_HID), lambda i: (0, 0)),
            pl.BlockSpec((D_IN, D_HID), lambda i: (0, 0)),
        ],
        out_specs=[
            pl.BlockSpec((R, D_HID), lambda i: (i, 0)),
            pl.BlockSpec((R, DP), lambda i: (i, 0)),
        ],
        out_shape=[
            jax.ShapeDtypeStruct((N_pad, D_HID), F32),
            jax.ShapeDtypeStruct((N_pad, DP), BF16),
        ],
    )


def _make_tc2(N_pad, R, D_HID, D_RNN, N_CLS, DP):
    NB = N_pad // R
    D = D_HID // 2

    def body(xw0_ref, agg_ref, degp_ref, bch_ref, batch_ref,
             wih_ref, whh_ref, bih_ref, bhh_ref, wfc_ref, bfc_ref,
             out_ref, pooled_acc, pre_scr, outs_scr):
        i = pl.program_id(0)

        @pl.when(i == 0)
        def _init():
            pooled_acc[...] = jnp.zeros_like(pooled_acc)

        deg = degp_ref[0] + degp_ref[1]
        dis = jnp.where(deg > 0.0,
                        lax.rsqrt(jnp.maximum(deg, 1e-30)), 0.0)
        ag = agg_ref[0].astype(F32) + agg_ref[1].astype(F32)
        agg = ag[:, :D_HID]
        h = xw0_ref[...] + bch_ref[...] - dis * agg
        h = jnp.maximum(h, 0.0)

        bids = batch_ref[...]
        onehot = (bids == lax.broadcasted_iota(I32, (R, B_GRAPHS), 1)
                  ).astype(F32)
        pooled_acc[...] += lax.dot_general(
            onehot, h, (((0,), (0,)), ((), ())), preferred_element_type=F32)

        @pl.when(i == NB - 1)
        def _tail():
            pooled = pooled_acc[...]
            pre = lax.dot_general(
                pooled, wih_ref[...], (((1,), (1,)), ((), ())),
                preferred_element_type=F32) + bih_ref[...] + bhh_ref[...]
            pre_scr[...] = pre

            def step(t, hprev):
                z = pre_scr[pl.ds(t, 1), :] + lax.dot_general(
                    hprev, whh_ref[...], (((1,), (1,)), ((), ())),
                    preferred_element_type=F32)
                hn = jnp.tanh(z)
                outs_scr[pl.ds(t, 1), :] = hn
                return hn
            lax.fori_loop(0, B_GRAPHS, step, jnp.zeros((1, D_RNN), F32))

            logits = lax.dot_general(
                outs_scr[...], wfc_ref[...], (((1,), (1,)), ((), ())),
                preferred_element_type=F32) + bfc_ref[...]
            m = jnp.max(logits, axis=1, keepdims=True)
            sh = logits - m
            out_ref[...] = sh - jnp.log(
                jnp.sum(jnp.exp(sh), axis=1, keepdims=True))

    return pl.pallas_call(
        body,
        grid=(NB,),
        in_specs=[
            pl.BlockSpec((R, D_HID), lambda i: (i, 0)),
            pl.BlockSpec((2, R, DP), lambda i: (0, i, 0)),
            pl.BlockSpec((2, R, 1), lambda i: (0, i, 0)),
            pl.BlockSpec((1, D_HID), lambda i: (0, 0)),
            pl.BlockSpec((R, 1), lambda i: (i, 0)),
            pl.BlockSpec((D_RNN, D_HID), lambda i: (0, 0)),
            pl.BlockSpec((D_RNN, D_RNN), lambda i: (0, 0)),
            pl.BlockSpec((1, D_RNN), lambda i: (0, 0)),
            pl.BlockSpec((1, D_RNN), lambda i: (0, 0)),
            pl.BlockSpec((N_CLS, D_RNN), lambda i: (0, 0)),
            pl.BlockSpec((1, N_CLS), lambda i: (0, 0)),
        ],
        out_specs=pl.BlockSpec((B_GRAPHS, N_CLS), lambda i: (0, 0)),
        out_shape=jax.ShapeDtypeStruct((B_GRAPHS, N_CLS), F32),
        scratch_shapes=[
            pltpu.VMEM((B_GRAPHS, D_HID), F32),
            pltpu.VMEM((B_GRAPHS, D_RNN), F32),
            pltpu.VMEM((B_GRAPHS, D_RNN), F32),
        ],
    )


def kernel(x, edge_index, edge_weight, batch, batch_size,
           W0, W1, b_cheb, W_ih, W_hh, b_ih, b_hh, W_fc, b_fc):
    N, D_IN = x.shape
    E = edge_index.shape[1]
    D_HID = W0.shape[1]
    D_RNN = W_ih.shape[0]
    N_CLS = W_fc.shape[0]
    DP = 64

    R = 1024
    N_pad = -(-N // R) * R
    n_rows = -(-E // (128 * NC * NS * 6)) * (NC * NS * 6)
    E_pad = n_rows * 128

    row = edge_index[0].astype(I32)
    col = edge_index[1].astype(I32)
    pad_e = E_pad - E
    row_p = jnp.pad(row, (0, pad_e)).reshape(n_rows, 128)
    col_p = jnp.pad(col, (0, pad_e)).reshape(n_rows, 128)
    ew_p = jnp.pad(edge_weight, (0, pad_e)).reshape(n_rows, 128)
    ew_bits = lax.bitcast_convert_type(ew_p, I32)
    cmb = jnp.stack([row_p, col_p, ew_bits], axis=1)

    degp = _make_deg_kernel(n_rows, N_pad)(cmb)
    degp3 = degp.reshape(NC, N_pad, 1)

    x_p = jnp.pad(x, ((0, N_pad - N), (0, 0)))
    xw0, tbl = _make_tc1(N_pad, R, D_IN, D_HID, DP)(x_p, degp3, W0, W1)

    aggs = _make_agg_kernel(n_rows, N_pad, DP)(cmb, tbl)

    batch_p = jnp.pad(batch.astype(I32), (0, N_pad - N),
                      constant_values=B_GRAPHS).reshape(N_pad, 1)
    out = _make_tc2(N_pad, R, D_HID, D_RNN, N_CLS, DP)(
        xw0, aggs, degp3, b_cheb.reshape(1, D_HID), batch_p,
        W_ih, W_hh, b_ih.reshape(1, D_RNN), b_hh.reshape(1, D_RNN),
        W_fc, b_fc.reshape(1, N_CLS))
    return out
